# baseline (speedup 1.0000x reference)
import jax
import jax.numpy as jnp
from jax import lax
from jax.experimental import pallas as pl
from jax.experimental.pallas import tpu as pltpu
from jax.experimental.pallas import tpu_sc as plsc

B = 4096
D = 64
N = 100000
L = 16
NC = 2
NS = 16
NW = NC * NS
BPW = B // NW

UB = 8192
RG = (N + UB - 1) // UB


def _repack_body(u_ref, i_ref, ub_ref, ib_ref, o_ref, ubf_ref, ibf_ref):
    o_ref[:, 0:D] = jnp.transpose(u_ref[...])
    o_ref[:, D:2 * D] = jnp.transpose(i_ref[...])
    ubf_ref[...] = ub_ref[0, :]
    ibf_ref[...] = ib_ref[0, :]


@jax.jit
def _tc_repack(ut, it, ubt, ibt):
    return pl.pallas_call(
        _repack_body,
        grid=(RG,),
        in_specs=[
            pl.BlockSpec((D, UB), lambda g: (0, g)),
            pl.BlockSpec((D, UB), lambda g: (0, g)),
            pl.BlockSpec((1, UB), lambda g: (0, g)),
            pl.BlockSpec((1, UB), lambda g: (0, g)),
        ],
        out_specs=[
            pl.BlockSpec((UB, 2 * D), lambda g: (g, 0)),
            pl.BlockSpec((UB,), lambda g: (g,)),
            pl.BlockSpec((UB,), lambda g: (g,)),
        ],
        out_shape=[
            jax.ShapeDtypeStruct((N, 2 * D), jnp.float32),
            jax.ShapeDtypeStruct((N,), jnp.float32),
            jax.ShapeDtypeStruct((N,), jnp.float32),
        ],
    )(ut, it, ubt, ibt)


def _sc_body(user_hbm, item_hbm, combo_hbm, ub_hbm, ib_hbm,
             dot_hbm, c_hbm,
             idx_u, idx_i, ubuf, ibuf, bub, bib, accv, cbuf, sem, semb):
    cid = lax.axis_index("c")
    sid = lax.axis_index("s")
    wid = sid * NC + cid
    base = wid * BPW

    pltpu.sync_copy(user_hbm.at[pl.ds(base, BPW)], idx_u)
    pltpu.sync_copy(item_hbm.at[pl.ds(base, BPW)], idx_i)

    cp_ub = pltpu.async_copy(ub_hbm.at[idx_u], bub, semb)
    cp_ib = pltpu.async_copy(ib_hbm.at[idx_i], bib, semb)

    cps = [
        pltpu.async_copy(combo_hbm.at[idx_u], ubuf, sem),
        pltpu.async_copy(combo_hbm.at[idx_i], ibuf, sem),
    ]
    for cp in cps:
        cp.wait()

    lanes = lax.iota(jnp.int32, L)
    for c in range(BPW // L):
        acc = jnp.zeros((L,), jnp.float32)
        for l in range(L):
            s = c * L + l
            p = ubuf[s, pl.ds(0, L)] * ibuf[s, pl.ds(D, L)]
            for k in range(1, D // L):
                p = p + ubuf[s, pl.ds(k * L, L)] * ibuf[s, pl.ds(D + k * L, L)]
            acc = jnp.where(lanes == l, jnp.sum(p), acc)
        accv[pl.ds(c * L, L)] = acc

    pltpu.sync_copy(accv, dot_hbm.at[pl.ds(base, BPW)])

    cp_ub.wait()
    cp_ib.wait()
    for c in range(BPW // L):
        s = pl.ds(c * L, L)
        cbuf[s] = bub[s] + bib[s]
    pltpu.sync_copy(cbuf, c_hbm.at[pl.ds(base, BPW)])


@jax.jit
def _sc_gather_dot(user, item, combo, ub_flat, ib_flat):
    mesh = plsc.VectorSubcoreMesh(core_axis_name="c", subcore_axis_name="s")
    return pl.kernel(
        _sc_body,
        mesh=mesh,
        compiler_params=pltpu.CompilerParams(
            use_tc_tiling_on_sc=True, needs_layout_passes=False),
        out_type=(
            jax.ShapeDtypeStruct((B,), jnp.float32),
            jax.ShapeDtypeStruct((B,), jnp.float32),
        ),
        scratch_types=[
            pltpu.VMEM((BPW,), jnp.int32),
            pltpu.VMEM((BPW,), jnp.int32),
            pltpu.VMEM((BPW, 2 * D), jnp.float32),
            pltpu.VMEM((BPW, 2 * D), jnp.float32),
            pltpu.VMEM((BPW,), jnp.float32),
            pltpu.VMEM((BPW,), jnp.float32),
            pltpu.VMEM((BPW,), jnp.float32),
            pltpu.VMEM((BPW,), jnp.float32),
            pltpu.SemaphoreType.DMA,
            pltpu.SemaphoreType.DMA,
        ],
    )(user, item, combo, ub_flat, ib_flat)


def _bcast_body(c_ref, r_ref, o_ref, c_col):
    @pl.when(pl.program_id(0) == 0)
    def _():
        c_col[...] = jnp.transpose(c_ref[...])

    i = pl.program_id(0)
    o_ref[...] = jnp.maximum(c_col[pl.ds(i * BM, BM), :] + r_ref[...], 0.0)


BM = 128


@jax.jit
def _tc_broadcast(dot_row, c_col):
    return pl.pallas_call(
        _bcast_body,
        grid=(B // BM,),
        in_specs=[
            pl.BlockSpec((1, B), lambda i: (0, 0)),
            pl.BlockSpec((1, B), lambda i: (0, 0)),
        ],
        out_specs=pl.BlockSpec((BM, B), lambda i: (i, 0)),
        out_shape=jax.ShapeDtypeStruct((B, B), jnp.float32),
        scratch_shapes=[pltpu.VMEM((B, 1), jnp.float32)],
    )(c_col, dot_row)


def kernel(user, item, user_emb, item_emb, user_bias_table, item_bias_table):
    combo, ubf, ibf = _tc_repack(user_emb.T, item_emb.T,
                                 user_bias_table.T, item_bias_table.T)
    dot, c = _sc_gather_dot(user, item, combo, ubf, ibf)
    return _tc_broadcast(dot.reshape(1, B), c.reshape(1, B))


# speedup vs baseline: 1.0503x; 1.0503x over previous
import jax
import jax.numpy as jnp
from jax import lax
from jax.experimental import pallas as pl
from jax.experimental.pallas import tpu as pltpu
from jax.experimental.pallas import tpu_sc as plsc

B = 4096
D = 64
N = 100000
L = 16
NC = 2
NS = 16
NW = NC * NS
BPW = B // NW

UB = 8192
RG = (N + UB - 1) // UB


def _repack_body(u_ref, i_ref, ub_ref, ib_ref, o_ref, ubf_ref, ibf_ref):
    o_ref[:, 0:D] = jnp.transpose(u_ref[...])
    o_ref[:, D:2 * D] = jnp.transpose(i_ref[...])
    ubf_ref[...] = ub_ref[0, :]
    ibf_ref[...] = ib_ref[0, :]


@jax.jit
def _tc_repack(ut, it, ubt, ibt):
    return pl.pallas_call(
        _repack_body,
        grid=(RG,),
        in_specs=[
            pl.BlockSpec((D, UB), lambda g: (0, g)),
            pl.BlockSpec((D, UB), lambda g: (0, g)),
            pl.BlockSpec((1, UB), lambda g: (0, g)),
            pl.BlockSpec((1, UB), lambda g: (0, g)),
        ],
        out_specs=[
            pl.BlockSpec((UB, 2 * D), lambda g: (g, 0)),
            pl.BlockSpec((UB,), lambda g: (g,)),
            pl.BlockSpec((UB,), lambda g: (g,)),
        ],
        out_shape=[
            jax.ShapeDtypeStruct((N, 2 * D), jnp.float32),
            jax.ShapeDtypeStruct((N,), jnp.float32),
            jax.ShapeDtypeStruct((N,), jnp.float32),
        ],
    )(ut, it, ubt, ibt)


def _sc_body(user_hbm, item_hbm, combo_hbm, ub_hbm, ib_hbm,
             dot_hbm, c_hbm,
             idx_u, idx_i, ubuf, ibuf, bub, bib, accv, cbuf, sem, semb):
    cid = lax.axis_index("c")
    sid = lax.axis_index("s")
    wid = sid * NC + cid
    base = wid * BPW

    pltpu.sync_copy(user_hbm.at[pl.ds(base, BPW)], idx_u)
    pltpu.sync_copy(item_hbm.at[pl.ds(base, BPW)], idx_i)

    cp_ub = pltpu.async_copy(ub_hbm.at[idx_u], bub, semb)
    cp_ib = pltpu.async_copy(ib_hbm.at[idx_i], bib, semb)

    cps = [
        pltpu.async_copy(combo_hbm.at[idx_u], ubuf, sem),
        pltpu.async_copy(combo_hbm.at[idx_i], ibuf, sem),
    ]
    for cp in cps:
        cp.wait()

    lanes = lax.iota(jnp.int32, L)
    for c in range(BPW // L):
        acc = jnp.zeros((L,), jnp.float32)
        for l in range(L):
            s = c * L + l
            p = ubuf[s, pl.ds(0, L)] * ibuf[s, pl.ds(D, L)]
            for k in range(1, D // L):
                p = p + ubuf[s, pl.ds(k * L, L)] * ibuf[s, pl.ds(D + k * L, L)]
            acc = jnp.where(lanes == l, jnp.sum(p), acc)
        accv[pl.ds(c * L, L)] = acc

    pltpu.sync_copy(accv, dot_hbm.at[pl.ds(base, BPW)])

    cp_ub.wait()
    cp_ib.wait()
    for c in range(BPW // L):
        s = pl.ds(c * L, L)
        cbuf[s] = bub[s] + bib[s]
    pltpu.sync_copy(cbuf, c_hbm.at[pl.ds(base, BPW)])


@jax.jit
def _sc_gather_dot(user, item, combo, ub_flat, ib_flat):
    mesh = plsc.VectorSubcoreMesh(core_axis_name="c", subcore_axis_name="s")
    return pl.kernel(
        _sc_body,
        mesh=mesh,
        compiler_params=pltpu.CompilerParams(
            use_tc_tiling_on_sc=True, needs_layout_passes=False),
        out_type=(
            jax.ShapeDtypeStruct((B,), jnp.float32),
            jax.ShapeDtypeStruct((B,), jnp.float32),
        ),
        scratch_types=[
            pltpu.VMEM((BPW,), jnp.int32),
            pltpu.VMEM((BPW,), jnp.int32),
            pltpu.VMEM((BPW, 2 * D), jnp.float32),
            pltpu.VMEM((BPW, 2 * D), jnp.float32),
            pltpu.VMEM((BPW,), jnp.float32),
            pltpu.VMEM((BPW,), jnp.float32),
            pltpu.VMEM((BPW,), jnp.float32),
            pltpu.VMEM((BPW,), jnp.float32),
            pltpu.SemaphoreType.DMA,
            pltpu.SemaphoreType.DMA,
        ],
    )(user, item, combo, ub_flat, ib_flat)


def _bcast_body(c_ref, r_ref, o_ref, c_col):
    @pl.when(pl.program_id(0) == 0)
    def _():
        c_col[...] = jnp.transpose(c_ref[...])

    i = pl.program_id(0)
    o_ref[...] = jnp.maximum(c_col[pl.ds(i * BM, BM), :] + r_ref[...], 0.0)


BM = 256


@jax.jit
def _tc_broadcast(dot_row, c_col):
    return pl.pallas_call(
        _bcast_body,
        grid=(B // BM,),
        in_specs=[
            pl.BlockSpec((1, B), lambda i: (0, 0)),
            pl.BlockSpec((1, B), lambda i: (0, 0)),
        ],
        out_specs=pl.BlockSpec((BM, B), lambda i: (i, 0)),
        out_shape=jax.ShapeDtypeStruct((B, B), jnp.float32),
        scratch_shapes=[pltpu.VMEM((B, 1), jnp.float32)],
    )(c_col, dot_row)


def kernel(user, item, user_emb, item_emb, user_bias_table, item_bias_table):
    combo, ubf, ibf = _tc_repack(user_emb.T, item_emb.T,
                                 user_bias_table.T, item_bias_table.T)
    dot, c = _sc_gather_dot(user, item, combo, ubf, ibf)
    return _tc_broadcast(dot.reshape(1, B), c.reshape(1, B))
